# baseline (speedup 1.0000x reference)
import functools

import jax
import jax.numpy as jnp
from jax import lax
from jax.experimental import pallas as pl
from jax.experimental.pallas import tpu as pltpu
from jax.experimental.pallas import tpu_sc as plsc

T = 4096
D = 1024
E = 8
O = 1024
BM = 128
NB = T // BM + E
PT = NB * BM
BR = 1024
NW = 32
TPT = T // NW
RCH = 32
XCH = TPT // RCH


def _router_body(x_ref, wr_ref, br_ref, probs_ref, routes_ref, ranks_ref,
                 counts_ref, eid_ref, counts_i_ref):
    m = pl.program_id(0)
    logits = jnp.dot(x_ref[...], wr_ref[...],
                     preferred_element_type=jnp.float32) + br_ref[...][None, :]
    mx = jnp.max(logits, axis=1, keepdims=True)
    ex = jnp.exp(logits - mx)
    probs = ex / jnp.sum(ex, axis=1, keepdims=True)
    probs_ref[...] = probs
    routes = jnp.argmax(probs, axis=1).astype(jnp.int32)
    routes_ref[...] = routes
    onehot = (lax.broadcasted_iota(jnp.int32, (BR, E), 1)
              == routes[:, None]).astype(jnp.float32)

    @pl.when(m == 0)
    def _():
        counts_ref[...] = jnp.zeros_like(counts_ref)

    carry = counts_ref[...][:E]
    rows = lax.broadcasted_iota(jnp.int32, (BR, BR), 0)
    cols = lax.broadcasted_iota(jnp.int32, (BR, BR), 1)
    tri = (cols < rows).astype(jnp.float32)
    local_excl = jnp.dot(tri, onehot, preferred_element_type=jnp.float32)
    rank_f = jnp.sum(onehot * (local_excl + carry[None, :]), axis=1)
    ranks_ref[...] = rank_f.astype(jnp.int32)
    counts_new = counts_ref[...] + jnp.pad(jnp.sum(onehot, axis=0),
                                           (0, 16 - E))
    counts_ref[...] = counts_new

    @pl.when(m == T // BR - 1)
    def _():
        counts_i_ref[...] = counts_new.astype(jnp.int32)

    @pl.when(m == T // BR - 1)
    def _():
        cfin = counts_new[:E]
        pblk = jnp.floor((cfin + (BM - 1)) * (1.0 / BM))
        tri8 = (lax.broadcasted_iota(jnp.int32, (E, E), 0)
                < lax.broadcasted_iota(jnp.int32, (E, E), 1)).astype(
                    jnp.float32)
        bb = jnp.dot(pblk[None, :], tri8,
                     preferred_element_type=jnp.float32)
        blk = lax.broadcasted_iota(jnp.int32, (48, E), 0).astype(jnp.float32)
        acc = jnp.sum((blk >= bb).astype(jnp.int32), axis=1)
        eid_ref[...] = jnp.clip(acc - 1, 0, E - 1)


def _router(x, Wr, br):
    return pl.pallas_call(
        _router_body,
        grid=(T // BR,),
        in_specs=[
            pl.BlockSpec((BR, D), lambda m: (m, 0)),
            pl.BlockSpec((D, E), lambda m: (0, 0)),
            pl.BlockSpec((E,), lambda m: (0,)),
        ],
        out_specs=[
            pl.BlockSpec((BR, E), lambda m: (m, 0)),
            pl.BlockSpec((BR,), lambda m: (m,)),
            pl.BlockSpec((BR,), lambda m: (m,)),
            pl.BlockSpec((16,), lambda m: (0,)),
            pl.BlockSpec((48,), lambda m: (0,)),
            pl.BlockSpec((16,), lambda m: (0,)),
        ],
        out_shape=[
            jax.ShapeDtypeStruct((T, E), jnp.float32),
            jax.ShapeDtypeStruct((T,), jnp.int32),
            jax.ShapeDtypeStruct((T,), jnp.int32),
            jax.ShapeDtypeStruct((16,), jnp.float32),
            jax.ShapeDtypeStruct((48,), jnp.int32),
            jax.ShapeDtypeStruct((16,), jnp.int32),
        ],
    )(x, Wr, br)


def _sc_mesh():
    return plsc.VectorSubcoreMesh(core_axis_name="c", subcore_axis_name="s",
                                  num_cores=2, num_subcores=16)


def _sc_params():
    return pltpu.CompilerParams(needs_layout_passes=False)


def _dest_slots(counts_hbm, routes_hbm, ranks_hbm, off_v, rt_v, rk_v, dst2,
                base_tok):
    lane = lax.broadcasted_iota(jnp.int32, (16,), 0)
    pltpu.sync_copy(counts_hbm, off_v)
    cnt = off_v[...]
    padded = jnp.where(lane < E, ((cnt + BM - 1) >> 7) << 7, 0)
    acc = padded
    for d in (1, 2, 4, 8):
        off_v[...] = acc
        g = plsc.load_gather(off_v, [jnp.maximum(lane - d, 0)])
        acc = acc + jnp.where(lane >= d, g, 0)
    off_v[...] = acc
    g = plsc.load_gather(off_v, [jnp.maximum(lane - 1, 0)])
    off_v[...] = jnp.where(lane >= 1, g, 0)
    pltpu.sync_copy(routes_hbm.at[pl.ds(base_tok, TPT)], rt_v)
    pltpu.sync_copy(ranks_hbm.at[pl.ds(base_tok, TPT)], rk_v)
    for j in range(TPT // 16):
        r = rt_v[pl.ds(16 * j, 16)]
        rk = rk_v[pl.ds(16 * j, 16)]
        dst2[j // 2, pl.ds((j % 2) * 16, 16)] = (
            plsc.load_gather(off_v, [r]) + rk)


@functools.cache
def _build_scatter_x():
    return functools.partial(
        pl.kernel,
        mesh=_sc_mesh(),
        compiler_params=_sc_params(),
        out_type=jax.ShapeDtypeStruct((PT, D), jnp.float32),
        scratch_types=[
            pltpu.VMEM((16,), jnp.int32),
            pltpu.VMEM((TPT,), jnp.int32),
            pltpu.VMEM((TPT,), jnp.int32),
            pltpu.VMEM((XCH, RCH), jnp.int32),
            pltpu.VMEM((RCH, D), jnp.float32),
            pltpu.VMEM((RCH, D), jnp.float32),
            pltpu.SemaphoreType.DMA,
            pltpu.SemaphoreType.DMA,
            pltpu.SemaphoreType.DMA,
            pltpu.SemaphoreType.DMA,
        ],
    )(_scatter_x_body)


def _scatter_x_body(routes_hbm, ranks_hbm, counts_hbm, x_hbm, xs_hbm,
                    off_v, rt_v, rk_v, dst2, rows0, rows1,
                    sin0, sin1, sout0, sout1):
    c = lax.axis_index("c")
    s = lax.axis_index("s")
    base_tok = (c * 16 + s) * TPT
    _dest_slots(counts_hbm, routes_hbm, ranks_hbm, off_v, rt_v, rk_v, dst2,
                base_tok)
    bufs = (rows0, rows1)
    sins = (sin0, sin1)
    souts = (sout0, sout1)
    loads = [None] * XCH
    stores = [None] * XCH
    loads[0] = pltpu.async_copy(x_hbm.at[pl.ds(base_tok, RCH)], rows0, sin0)
    for j in range(XCH):
        loads[j].wait()
        if j + 1 < XCH:
            if j >= 1:
                stores[j - 1].wait()
            loads[j + 1] = pltpu.async_copy(
                x_hbm.at[pl.ds(base_tok + RCH * (j + 1), RCH)],
                bufs[(j + 1) % 2], sins[(j + 1) % 2])
        stores[j] = pltpu.async_copy(bufs[j % 2], xs_hbm.at[dst2.at[j]],
                                     souts[j % 2])
    stores[XCH - 2].wait()
    stores[XCH - 1].wait()


def _mm_body(eid_ref, x_ref, w_ref, b_ref, y_ref):
    y_ref[...] = jnp.dot(x_ref[...], w_ref[0],
                         preferred_element_type=jnp.float32) + b_ref[0]


def _grouped_mm(eid, xs, W, b):
    grid_spec = pltpu.PrefetchScalarGridSpec(
        num_scalar_prefetch=1,
        grid=(NB,),
        in_specs=[
            pl.BlockSpec((BM, D), lambda m, eid_ref: (m, 0)),
            pl.BlockSpec((1, D, O), lambda m, eid_ref: (eid_ref[m], 0, 0)),
            pl.BlockSpec((1, 1, O), lambda m, eid_ref: (eid_ref[m], 0, 0)),
        ],
        out_specs=pl.BlockSpec((BM, O), lambda m, eid_ref: (m, 0)),
    )
    return pl.pallas_call(
        _mm_body,
        grid_spec=grid_spec,
        out_shape=jax.ShapeDtypeStruct((PT, O), jnp.float32),
    )(eid, xs, W, b.reshape(E, 1, O))


@functools.cache
def _build_gather_y():
    return functools.partial(
        pl.kernel,
        mesh=_sc_mesh(),
        compiler_params=_sc_params(),
        out_type=jax.ShapeDtypeStruct((T, O), jnp.float32),
        scratch_types=[
            pltpu.VMEM((16,), jnp.int32),
            pltpu.VMEM((TPT,), jnp.int32),
            pltpu.VMEM((TPT,), jnp.int32),
            pltpu.VMEM((XCH, RCH), jnp.int32),
            pltpu.VMEM((RCH, O), jnp.float32),
            pltpu.VMEM((RCH, O), jnp.float32),
            pltpu.SemaphoreType.DMA,
            pltpu.SemaphoreType.DMA,
            pltpu.SemaphoreType.DMA,
            pltpu.SemaphoreType.DMA,
        ],
    )(_gather_y_body)


def _gather_y_body(routes_hbm, ranks_hbm, counts_hbm, ys_hbm, out_hbm,
                   off_v, rt_v, rk_v, dst2, rows0, rows1,
                   sin0, sin1, sout0, sout1):
    c = lax.axis_index("c")
    s = lax.axis_index("s")
    base_tok = (c * 16 + s) * TPT
    _dest_slots(counts_hbm, routes_hbm, ranks_hbm, off_v, rt_v, rk_v, dst2,
                base_tok)
    bufs = (rows0, rows1)
    sins = (sin0, sin1)
    souts = (sout0, sout1)
    loads = [None] * XCH
    stores = [None] * XCH
    loads[0] = pltpu.async_copy(ys_hbm.at[dst2.at[0]], rows0, sin0)
    for j in range(XCH):
        loads[j].wait()
        if j + 1 < XCH:
            if j >= 1:
                stores[j - 1].wait()
            loads[j + 1] = pltpu.async_copy(ys_hbm.at[dst2.at[j + 1]],
                                            bufs[(j + 1) % 2],
                                            sins[(j + 1) % 2])
        stores[j] = pltpu.async_copy(
            bufs[j % 2], out_hbm.at[pl.ds(base_tok + RCH * j, RCH)],
            souts[j % 2])
    stores[XCH - 2].wait()
    stores[XCH - 1].wait()


def kernel(x, Wr, br, W, b):
    probs, routes, ranks, counts16, eid, counts_i = _router(x, Wr, br)
    xs = _build_scatter_x()(routes, ranks, counts_i, x)
    ys = _grouped_mm(eid, xs, W, b)
    out = _build_gather_y()(routes, ranks, counts_i, ys)
    counts = counts16[:E]
    return (out, out, probs, counts)


# speedup vs baseline: 1.0307x; 1.0307x over previous
import functools

import jax
import jax.numpy as jnp
from jax import lax
from jax.experimental import pallas as pl
from jax.experimental.pallas import tpu as pltpu
from jax.experimental.pallas import tpu_sc as plsc

T = 4096
D = 1024
E = 8
O = 1024
BM = 128
NB = T // BM + E
PT = NB * BM
BR = 512
NW = 32
TPT = T // NW
RCH = 32
XCH = TPT // RCH


def _router_body(x_ref, wr_ref, br_ref, probs_ref, routes_ref, ranks_ref,
                 counts_ref, eid_ref, counts_i_ref):
    m = pl.program_id(0)
    logits = jnp.dot(x_ref[...], wr_ref[...],
                     preferred_element_type=jnp.float32) + br_ref[...][None, :]
    mx = jnp.max(logits, axis=1, keepdims=True)
    ex = jnp.exp(logits - mx)
    probs = ex / jnp.sum(ex, axis=1, keepdims=True)
    probs_ref[...] = probs
    routes = jnp.argmax(probs, axis=1).astype(jnp.int32)
    routes_ref[...] = routes
    onehot = (lax.broadcasted_iota(jnp.int32, (BR, E), 1)
              == routes[:, None]).astype(jnp.float32)

    @pl.when(m == 0)
    def _():
        counts_ref[...] = jnp.zeros_like(counts_ref)

    carry = counts_ref[...][:E]
    rows = lax.broadcasted_iota(jnp.int32, (BR, BR), 0)
    cols = lax.broadcasted_iota(jnp.int32, (BR, BR), 1)
    tri = (cols < rows).astype(jnp.float32)
    local_excl = jnp.dot(tri, onehot, preferred_element_type=jnp.float32)
    rank_f = jnp.sum(onehot * (local_excl + carry[None, :]), axis=1)
    ranks_ref[...] = rank_f.astype(jnp.int32)
    counts_new = counts_ref[...] + jnp.pad(jnp.sum(onehot, axis=0),
                                           (0, 16 - E))
    counts_ref[...] = counts_new

    @pl.when(m == T // BR - 1)
    def _():
        counts_i_ref[...] = counts_new.astype(jnp.int32)

    @pl.when(m == T // BR - 1)
    def _():
        cfin = counts_new[:E]
        pblk = jnp.floor((cfin + (BM - 1)) * (1.0 / BM))
        tri8 = (lax.broadcasted_iota(jnp.int32, (E, E), 0)
                < lax.broadcasted_iota(jnp.int32, (E, E), 1)).astype(
                    jnp.float32)
        bb = jnp.dot(pblk[None, :], tri8,
                     preferred_element_type=jnp.float32)
        blk = lax.broadcasted_iota(jnp.int32, (48, E), 0).astype(jnp.float32)
        acc = jnp.sum((blk >= bb).astype(jnp.int32), axis=1)
        eid_ref[...] = jnp.clip(acc - 1, 0, E - 1)


def _router(x, Wr, br):
    return pl.pallas_call(
        _router_body,
        grid=(T // BR,),
        in_specs=[
            pl.BlockSpec((BR, D), lambda m: (m, 0)),
            pl.BlockSpec((D, E), lambda m: (0, 0)),
            pl.BlockSpec((E,), lambda m: (0,)),
        ],
        out_specs=[
            pl.BlockSpec((BR, E), lambda m: (m, 0)),
            pl.BlockSpec((BR,), lambda m: (m,)),
            pl.BlockSpec((BR,), lambda m: (m,)),
            pl.BlockSpec((16,), lambda m: (0,)),
            pl.BlockSpec((48,), lambda m: (0,)),
            pl.BlockSpec((16,), lambda m: (0,)),
        ],
        out_shape=[
            jax.ShapeDtypeStruct((T, E), jnp.float32),
            jax.ShapeDtypeStruct((T,), jnp.int32),
            jax.ShapeDtypeStruct((T,), jnp.int32),
            jax.ShapeDtypeStruct((16,), jnp.float32),
            jax.ShapeDtypeStruct((48,), jnp.int32),
            jax.ShapeDtypeStruct((16,), jnp.int32),
        ],
    )(x, Wr, br)


def _sc_mesh():
    return plsc.VectorSubcoreMesh(core_axis_name="c", subcore_axis_name="s",
                                  num_cores=2, num_subcores=16)


def _sc_params():
    return pltpu.CompilerParams(needs_layout_passes=False)


def _dest_slots(counts_hbm, routes_hbm, ranks_hbm, off_v, rt_v, rk_v, dst2,
                base_tok):
    lane = lax.broadcasted_iota(jnp.int32, (16,), 0)
    pltpu.sync_copy(counts_hbm, off_v)
    cnt = off_v[...]
    padded = jnp.where(lane < E, ((cnt + BM - 1) >> 7) << 7, 0)
    acc = padded
    for d in (1, 2, 4, 8):
        off_v[...] = acc
        g = plsc.load_gather(off_v, [jnp.maximum(lane - d, 0)])
        acc = acc + jnp.where(lane >= d, g, 0)
    off_v[...] = acc
    g = plsc.load_gather(off_v, [jnp.maximum(lane - 1, 0)])
    off_v[...] = jnp.where(lane >= 1, g, 0)
    pltpu.sync_copy(routes_hbm.at[pl.ds(base_tok, TPT)], rt_v)
    pltpu.sync_copy(ranks_hbm.at[pl.ds(base_tok, TPT)], rk_v)
    for j in range(TPT // 16):
        r = rt_v[pl.ds(16 * j, 16)]
        rk = rk_v[pl.ds(16 * j, 16)]
        dst2[j // 2, pl.ds((j % 2) * 16, 16)] = (
            plsc.load_gather(off_v, [r]) + rk)


@functools.cache
def _build_scatter_x():
    return functools.partial(
        pl.kernel,
        mesh=_sc_mesh(),
        compiler_params=_sc_params(),
        out_type=jax.ShapeDtypeStruct((PT, D), jnp.float32),
        scratch_types=[
            pltpu.VMEM((16,), jnp.int32),
            pltpu.VMEM((TPT,), jnp.int32),
            pltpu.VMEM((TPT,), jnp.int32),
            pltpu.VMEM((XCH, RCH), jnp.int32),
            pltpu.VMEM((RCH, D), jnp.float32),
            pltpu.VMEM((RCH, D), jnp.float32),
            pltpu.SemaphoreType.DMA,
            pltpu.SemaphoreType.DMA,
            pltpu.SemaphoreType.DMA,
            pltpu.SemaphoreType.DMA,
        ],
    )(_scatter_x_body)


def _scatter_x_body(routes_hbm, ranks_hbm, counts_hbm, x_hbm, xs_hbm,
                    off_v, rt_v, rk_v, dst2, rows0, rows1,
                    sin0, sin1, sout0, sout1):
    c = lax.axis_index("c")
    s = lax.axis_index("s")
    base_tok = (c * 16 + s) * TPT
    _dest_slots(counts_hbm, routes_hbm, ranks_hbm, off_v, rt_v, rk_v, dst2,
                base_tok)
    bufs = (rows0, rows1)
    sins = (sin0, sin1)
    souts = (sout0, sout1)
    loads = [None] * XCH
    stores = [None] * XCH
    loads[0] = pltpu.async_copy(x_hbm.at[pl.ds(base_tok, RCH)], rows0, sin0)
    for j in range(XCH):
        loads[j].wait()
        if j + 1 < XCH:
            if j >= 1:
                stores[j - 1].wait()
            loads[j + 1] = pltpu.async_copy(
                x_hbm.at[pl.ds(base_tok + RCH * (j + 1), RCH)],
                bufs[(j + 1) % 2], sins[(j + 1) % 2])
        stores[j] = pltpu.async_copy(bufs[j % 2], xs_hbm.at[dst2.at[j]],
                                     souts[j % 2])
    stores[XCH - 2].wait()
    stores[XCH - 1].wait()


def _mm_body(eid_ref, x_ref, w_ref, b_ref, y_ref):
    y_ref[...] = jnp.dot(x_ref[...], w_ref[0],
                         preferred_element_type=jnp.float32) + b_ref[0]


def _grouped_mm(eid, xs, W, b):
    grid_spec = pltpu.PrefetchScalarGridSpec(
        num_scalar_prefetch=1,
        grid=(NB,),
        in_specs=[
            pl.BlockSpec((BM, D), lambda m, eid_ref: (m, 0)),
            pl.BlockSpec((1, D, O), lambda m, eid_ref: (eid_ref[m], 0, 0)),
            pl.BlockSpec((1, 1, O), lambda m, eid_ref: (eid_ref[m], 0, 0)),
        ],
        out_specs=pl.BlockSpec((BM, O), lambda m, eid_ref: (m, 0)),
    )
    return pl.pallas_call(
        _mm_body,
        grid_spec=grid_spec,
        out_shape=jax.ShapeDtypeStruct((PT, O), jnp.float32),
    )(eid, xs, W, b.reshape(E, 1, O))


@functools.cache
def _build_gather_y():
    return functools.partial(
        pl.kernel,
        mesh=_sc_mesh(),
        compiler_params=_sc_params(),
        out_type=jax.ShapeDtypeStruct((T, O), jnp.float32),
        scratch_types=[
            pltpu.VMEM((16,), jnp.int32),
            pltpu.VMEM((TPT,), jnp.int32),
            pltpu.VMEM((TPT,), jnp.int32),
            pltpu.VMEM((XCH, RCH), jnp.int32),
            pltpu.VMEM((RCH, O), jnp.float32),
            pltpu.VMEM((RCH, O), jnp.float32),
            pltpu.SemaphoreType.DMA,
            pltpu.SemaphoreType.DMA,
            pltpu.SemaphoreType.DMA,
            pltpu.SemaphoreType.DMA,
        ],
    )(_gather_y_body)


def _gather_y_body(routes_hbm, ranks_hbm, counts_hbm, ys_hbm, out_hbm,
                   off_v, rt_v, rk_v, dst2, rows0, rows1,
                   sin0, sin1, sout0, sout1):
    c = lax.axis_index("c")
    s = lax.axis_index("s")
    base_tok = (c * 16 + s) * TPT
    _dest_slots(counts_hbm, routes_hbm, ranks_hbm, off_v, rt_v, rk_v, dst2,
                base_tok)
    bufs = (rows0, rows1)
    sins = (sin0, sin1)
    souts = (sout0, sout1)
    loads = [None] * XCH
    stores = [None] * XCH
    loads[0] = pltpu.async_copy(ys_hbm.at[dst2.at[0]], rows0, sin0)
    for j in range(XCH):
        loads[j].wait()
        if j + 1 < XCH:
            if j >= 1:
                stores[j - 1].wait()
            loads[j + 1] = pltpu.async_copy(ys_hbm.at[dst2.at[j + 1]],
                                            bufs[(j + 1) % 2],
                                            sins[(j + 1) % 2])
        stores[j] = pltpu.async_copy(
            bufs[j % 2], out_hbm.at[pl.ds(base_tok + RCH * j, RCH)],
            souts[j % 2])
    stores[XCH - 2].wait()
    stores[XCH - 1].wait()


def kernel(x, Wr, br, W, b):
    probs, routes, ranks, counts16, eid, counts_i = _router(x, Wr, br)
    xs = _build_scatter_x()(routes, ranks, counts_i, x)
    ys = _grouped_mm(eid, xs, W, b)
    out = _build_gather_y()(routes, ranks, counts_i, ys)
    counts = counts16[:E]
    return (out, out, probs, counts)


# speedup vs baseline: 1.0497x; 1.0184x over previous
import functools

import jax
import jax.numpy as jnp
from jax import lax
from jax.experimental import pallas as pl
from jax.experimental.pallas import tpu as pltpu
from jax.experimental.pallas import tpu_sc as plsc

T = 4096
D = 1024
E = 8
O = 1024
BM = 128
NB = T // BM + E
PT = NB * BM
BR = 512
NW = 32
TPT = T // NW
RCH = 32
XCH = TPT // RCH


def _router_body(x_ref, wr_ref, br_ref, probs_ref, routes_ref, ranks_ref,
                 counts_ref, eid_ref, counts_i_ref):
    m = pl.program_id(0)
    logits = jnp.dot(x_ref[...], wr_ref[...],
                     preferred_element_type=jnp.float32) + br_ref[...][None, :]
    mx = jnp.max(logits, axis=1, keepdims=True)
    ex = jnp.exp(logits - mx)
    probs = ex / jnp.sum(ex, axis=1, keepdims=True)
    probs_ref[...] = probs
    routes = jnp.argmax(probs, axis=1).astype(jnp.int32)
    routes_ref[...] = routes
    onehot = (lax.broadcasted_iota(jnp.int32, (BR, E), 1)
              == routes[:, None]).astype(jnp.float32)

    @pl.when(m == 0)
    def _():
        counts_ref[...] = jnp.zeros_like(counts_ref)

    carry = counts_ref[...][:E]
    rows = lax.broadcasted_iota(jnp.int32, (BR, BR), 0)
    cols = lax.broadcasted_iota(jnp.int32, (BR, BR), 1)
    tri = (cols < rows).astype(jnp.float32)
    local_excl = jnp.dot(tri, onehot, preferred_element_type=jnp.float32)
    rank_f = jnp.sum(onehot * (local_excl + carry[None, :]), axis=1)
    ranks_ref[...] = rank_f.astype(jnp.int32)
    counts_new = counts_ref[...] + jnp.pad(jnp.sum(onehot, axis=0),
                                           (0, 16 - E))
    counts_ref[...] = counts_new

    @pl.when(m == T // BR - 1)
    def _():
        counts_i_ref[...] = counts_new.astype(jnp.int32)

    @pl.when(m == T // BR - 1)
    def _():
        cfin = counts_new[:E]
        pblk = jnp.floor((cfin + (BM - 1)) * (1.0 / BM))
        tri8 = (lax.broadcasted_iota(jnp.int32, (E, E), 0)
                < lax.broadcasted_iota(jnp.int32, (E, E), 1)).astype(
                    jnp.float32)
        bb = jnp.dot(pblk[None, :], tri8,
                     preferred_element_type=jnp.float32)
        blk = lax.broadcasted_iota(jnp.int32, (48, E), 0).astype(jnp.float32)
        acc = jnp.sum((blk >= bb).astype(jnp.int32), axis=1)
        eid_ref[...] = jnp.clip(acc - 1, 0, E - 1)


def _router(x, Wr, br):
    return pl.pallas_call(
        _router_body,
        grid=(T // BR,),
        in_specs=[
            pl.BlockSpec((BR, D), lambda m: (m, 0)),
            pl.BlockSpec((D, E), lambda m: (0, 0)),
            pl.BlockSpec((E,), lambda m: (0,)),
        ],
        out_specs=[
            pl.BlockSpec((BR, E), lambda m: (m, 0)),
            pl.BlockSpec((BR,), lambda m: (m,)),
            pl.BlockSpec((BR,), lambda m: (m,)),
            pl.BlockSpec((16,), lambda m: (0,)),
            pl.BlockSpec((48,), lambda m: (0,)),
            pl.BlockSpec((16,), lambda m: (0,)),
        ],
        out_shape=[
            jax.ShapeDtypeStruct((T, E), jnp.float32),
            jax.ShapeDtypeStruct((T,), jnp.int32),
            jax.ShapeDtypeStruct((T,), jnp.int32),
            jax.ShapeDtypeStruct((16,), jnp.float32),
            jax.ShapeDtypeStruct((48,), jnp.int32),
            jax.ShapeDtypeStruct((16,), jnp.int32),
        ],
    )(x, Wr, br)


def _sc_mesh():
    return plsc.VectorSubcoreMesh(core_axis_name="c", subcore_axis_name="s",
                                  num_cores=2, num_subcores=16)


def _sc_params():
    return pltpu.CompilerParams(needs_layout_passes=False)


def _dest_slots(counts_hbm, routes_hbm, ranks_hbm, off_v, rt_v, rk_v, dst2,
                base_tok):
    lane = lax.broadcasted_iota(jnp.int32, (16,), 0)
    pltpu.sync_copy(counts_hbm, off_v)
    cnt = off_v[...]
    padded = jnp.where(lane < E, ((cnt + BM - 1) >> 7) << 7, 0)
    acc = padded
    for d in (1, 2, 4, 8):
        off_v[...] = acc
        g = plsc.load_gather(off_v, [jnp.maximum(lane - d, 0)])
        acc = acc + jnp.where(lane >= d, g, 0)
    off_v[...] = acc
    g = plsc.load_gather(off_v, [jnp.maximum(lane - 1, 0)])
    off_v[...] = jnp.where(lane >= 1, g, 0)
    pltpu.sync_copy(routes_hbm.at[pl.ds(base_tok, TPT)], rt_v)
    pltpu.sync_copy(ranks_hbm.at[pl.ds(base_tok, TPT)], rk_v)
    for j in range(TPT // 16):
        r = rt_v[pl.ds(16 * j, 16)]
        rk = rk_v[pl.ds(16 * j, 16)]
        dst2[j // 2, pl.ds((j % 2) * 16, 16)] = (
            plsc.load_gather(off_v, [r]) + rk)


@functools.cache
def _build_scatter_x():
    return functools.partial(
        pl.kernel,
        mesh=_sc_mesh(),
        compiler_params=_sc_params(),
        out_type=jax.ShapeDtypeStruct((PT, D), jnp.float32),
        scratch_types=[
            pltpu.VMEM((16,), jnp.int32),
            pltpu.VMEM((TPT,), jnp.int32),
            pltpu.VMEM((TPT,), jnp.int32),
            pltpu.VMEM((XCH, RCH), jnp.int32),
            pltpu.VMEM((RCH, D), jnp.float32),
            pltpu.VMEM((RCH, D), jnp.float32),
            pltpu.SemaphoreType.DMA,
            pltpu.SemaphoreType.DMA,
            pltpu.SemaphoreType.DMA,
            pltpu.SemaphoreType.DMA,
        ],
    )(_scatter_x_body)


def _scatter_x_body(routes_hbm, ranks_hbm, counts_hbm, x_hbm, xs_hbm,
                    off_v, rt_v, rk_v, dst2, rows0, rows1,
                    sin0, sin1, sout0, sout1):
    c = lax.axis_index("c")
    s = lax.axis_index("s")
    base_tok = (c * 16 + s) * TPT
    _dest_slots(counts_hbm, routes_hbm, ranks_hbm, off_v, rt_v, rk_v, dst2,
                base_tok)
    bufs = (rows0, rows1)
    sins = (sin0, sin1)
    souts = (sout0, sout1)
    loads = [None] * XCH
    stores = [None] * XCH
    loads[0] = pltpu.async_copy(x_hbm.at[pl.ds(base_tok, RCH)], rows0, sin0)
    for j in range(XCH):
        loads[j].wait()
        if j + 1 < XCH:
            if j >= 1:
                stores[j - 1].wait()
            loads[j + 1] = pltpu.async_copy(
                x_hbm.at[pl.ds(base_tok + RCH * (j + 1), RCH)],
                bufs[(j + 1) % 2], sins[(j + 1) % 2])
        stores[j] = pltpu.async_copy(bufs[j % 2], xs_hbm.at[dst2.at[j]],
                                     souts[j % 2])
    stores[XCH - 2].wait()
    stores[XCH - 1].wait()


def _mm_body(eid_ref, x_ref, w_ref, b_ref, y_ref):
    e = eid_ref[pl.program_id(0)]
    onehot_e = (lax.broadcasted_iota(jnp.int32, (E,), 0)
                == e).astype(jnp.float32)
    b_row = jnp.sum(b_ref[...] * onehot_e[:, None], axis=0)
    y_ref[...] = jnp.dot(x_ref[...], w_ref[e],
                         preferred_element_type=jnp.float32) + b_row[None, :]


def _grouped_mm(eid, xs, W, b):
    grid_spec = pltpu.PrefetchScalarGridSpec(
        num_scalar_prefetch=1,
        grid=(NB,),
        in_specs=[
            pl.BlockSpec((BM, D), lambda m, eid_ref: (m, 0)),
            pl.BlockSpec((E, D, O), lambda m, eid_ref: (0, 0, 0)),
            pl.BlockSpec((E, O), lambda m, eid_ref: (0, 0)),
        ],
        out_specs=pl.BlockSpec((BM, O), lambda m, eid_ref: (m, 0)),
    )
    return pl.pallas_call(
        _mm_body,
        grid_spec=grid_spec,
        out_shape=jax.ShapeDtypeStruct((PT, O), jnp.float32),
    )(eid, xs, W, b)


@functools.cache
def _build_gather_y():
    return functools.partial(
        pl.kernel,
        mesh=_sc_mesh(),
        compiler_params=_sc_params(),
        out_type=jax.ShapeDtypeStruct((T, O), jnp.float32),
        scratch_types=[
            pltpu.VMEM((16,), jnp.int32),
            pltpu.VMEM((TPT,), jnp.int32),
            pltpu.VMEM((TPT,), jnp.int32),
            pltpu.VMEM((XCH, RCH), jnp.int32),
            pltpu.VMEM((RCH, O), jnp.float32),
            pltpu.VMEM((RCH, O), jnp.float32),
            pltpu.SemaphoreType.DMA,
            pltpu.SemaphoreType.DMA,
            pltpu.SemaphoreType.DMA,
            pltpu.SemaphoreType.DMA,
        ],
    )(_gather_y_body)


def _gather_y_body(routes_hbm, ranks_hbm, counts_hbm, ys_hbm, out_hbm,
                   off_v, rt_v, rk_v, dst2, rows0, rows1,
                   sin0, sin1, sout0, sout1):
    c = lax.axis_index("c")
    s = lax.axis_index("s")
    base_tok = (c * 16 + s) * TPT
    _dest_slots(counts_hbm, routes_hbm, ranks_hbm, off_v, rt_v, rk_v, dst2,
                base_tok)
    bufs = (rows0, rows1)
    sins = (sin0, sin1)
    souts = (sout0, sout1)
    loads = [None] * XCH
    stores = [None] * XCH
    loads[0] = pltpu.async_copy(ys_hbm.at[dst2.at[0]], rows0, sin0)
    for j in range(XCH):
        loads[j].wait()
        if j + 1 < XCH:
            if j >= 1:
                stores[j - 1].wait()
            loads[j + 1] = pltpu.async_copy(ys_hbm.at[dst2.at[j + 1]],
                                            bufs[(j + 1) % 2],
                                            sins[(j + 1) % 2])
        stores[j] = pltpu.async_copy(
            bufs[j % 2], out_hbm.at[pl.ds(base_tok + RCH * j, RCH)],
            souts[j % 2])
    stores[XCH - 2].wait()
    stores[XCH - 1].wait()


def kernel(x, Wr, br, W, b):
    probs, routes, ranks, counts16, eid, counts_i = _router(x, Wr, br)
    xs = _build_scatter_x()(routes, ranks, counts_i, x)
    ys = _grouped_mm(eid, xs, W, b)
    out = _build_gather_y()(routes, ranks, counts_i, ys)
    counts = counts16[:E]
    return (out, out, probs, counts)


# speedup vs baseline: 1.1192x; 1.0663x over previous
import functools

import jax
import jax.numpy as jnp
from jax import lax
from jax.experimental import pallas as pl
from jax.experimental.pallas import tpu as pltpu
from jax.experimental.pallas import tpu_sc as plsc

T = 4096
D = 1024
E = 8
O = 1024
BM = 128
NB = T // BM + E
PT = NB * BM
BR = 512
NW = 32
TPT = T // NW
RCH = 32
XCH = TPT // RCH


def _router_body(x_ref, wr_ref, br_ref, probs_ref, routes_ref, ranks_ref,
                 counts_ref, eid_ref, counts_i_ref):
    m = pl.program_id(0)
    logits = jnp.dot(x_ref[...], wr_ref[...],
                     preferred_element_type=jnp.float32) + br_ref[...][None, :]
    mx = jnp.max(logits, axis=1, keepdims=True)
    ex = jnp.exp(logits - mx)
    probs = ex / jnp.sum(ex, axis=1, keepdims=True)
    probs_ref[...] = probs
    routes = jnp.argmax(probs, axis=1).astype(jnp.int32)
    routes_ref[...] = routes
    onehot = (lax.broadcasted_iota(jnp.int32, (BR, E), 1)
              == routes[:, None]).astype(jnp.float32)

    @pl.when(m == 0)
    def _():
        counts_ref[...] = jnp.zeros_like(counts_ref)

    carry = counts_ref[...][:E]
    rows = lax.broadcasted_iota(jnp.int32, (BR, BR), 0)
    cols = lax.broadcasted_iota(jnp.int32, (BR, BR), 1)
    tri = (cols < rows).astype(jnp.float32)
    local_excl = jnp.dot(tri, onehot, preferred_element_type=jnp.float32)
    rank_f = jnp.sum(onehot * (local_excl + carry[None, :]), axis=1)
    ranks_ref[...] = rank_f.astype(jnp.int32)
    counts_new = counts_ref[...] + jnp.pad(jnp.sum(onehot, axis=0),
                                           (0, 16 - E))
    counts_ref[...] = counts_new

    @pl.when(m == T // BR - 1)
    def _():
        counts_i_ref[...] = counts_new.astype(jnp.int32)

    @pl.when(m == T // BR - 1)
    def _():
        cfin = counts_new[:E]
        pblk = jnp.floor((cfin + (BM - 1)) * (1.0 / BM))
        tri8 = (lax.broadcasted_iota(jnp.int32, (E, E), 0)
                < lax.broadcasted_iota(jnp.int32, (E, E), 1)).astype(
                    jnp.float32)
        bb = jnp.dot(pblk[None, :], tri8,
                     preferred_element_type=jnp.float32)
        blk = lax.broadcasted_iota(jnp.int32, (48, E), 0).astype(jnp.float32)
        acc = jnp.sum((blk >= bb).astype(jnp.int32), axis=1)
        eid_ref[...] = jnp.clip(acc - 1, 0, E - 1)


def _router(x, Wr, br):
    return pl.pallas_call(
        _router_body,
        grid=(T // BR,),
        in_specs=[
            pl.BlockSpec((BR, D), lambda m: (m, 0)),
            pl.BlockSpec((D, E), lambda m: (0, 0)),
            pl.BlockSpec((E,), lambda m: (0,)),
        ],
        out_specs=[
            pl.BlockSpec((BR, E), lambda m: (m, 0)),
            pl.BlockSpec((BR,), lambda m: (m,)),
            pl.BlockSpec((BR,), lambda m: (m,)),
            pl.BlockSpec((16,), lambda m: (0,)),
            pl.BlockSpec((48,), lambda m: (0,)),
            pl.BlockSpec((16,), lambda m: (0,)),
        ],
        out_shape=[
            jax.ShapeDtypeStruct((T, E), jnp.float32),
            jax.ShapeDtypeStruct((T,), jnp.int32),
            jax.ShapeDtypeStruct((T,), jnp.int32),
            jax.ShapeDtypeStruct((16,), jnp.float32),
            jax.ShapeDtypeStruct((48,), jnp.int32),
            jax.ShapeDtypeStruct((16,), jnp.int32),
        ],
    )(x, Wr, br)


def _sc_mesh():
    return plsc.VectorSubcoreMesh(core_axis_name="c", subcore_axis_name="s",
                                  num_cores=2, num_subcores=16)


def _sc_params():
    return pltpu.CompilerParams(needs_layout_passes=False)


def _dest_slots(counts_hbm, routes_hbm, ranks_hbm, off_v, rt_v, rk_v, dst2,
                base_tok):
    lane = lax.broadcasted_iota(jnp.int32, (16,), 0)
    pltpu.sync_copy(counts_hbm, off_v)
    cnt = off_v[...]
    padded = jnp.where(lane < E, ((cnt + BM - 1) >> 7) << 7, 0)
    acc = padded
    for d in (1, 2, 4, 8):
        off_v[...] = acc
        g = plsc.load_gather(off_v, [jnp.maximum(lane - d, 0)])
        acc = acc + jnp.where(lane >= d, g, 0)
    off_v[...] = acc
    g = plsc.load_gather(off_v, [jnp.maximum(lane - 1, 0)])
    off_v[...] = jnp.where(lane >= 1, g, 0)
    pltpu.sync_copy(routes_hbm.at[pl.ds(base_tok, TPT)], rt_v)
    pltpu.sync_copy(ranks_hbm.at[pl.ds(base_tok, TPT)], rk_v)
    for j in range(TPT // 16):
        r = rt_v[pl.ds(16 * j, 16)]
        rk = rk_v[pl.ds(16 * j, 16)]
        dst2[j // 2, pl.ds((j % 2) * 16, 16)] = (
            plsc.load_gather(off_v, [r]) + rk)


@functools.cache
def _build_scatter_x():
    return functools.partial(
        pl.kernel,
        mesh=_sc_mesh(),
        compiler_params=_sc_params(),
        out_type=jax.ShapeDtypeStruct((PT, D), jnp.float32),
        scratch_types=[
            pltpu.VMEM((16,), jnp.int32),
            pltpu.VMEM((TPT,), jnp.int32),
            pltpu.VMEM((TPT,), jnp.int32),
            pltpu.VMEM((XCH, RCH), jnp.int32),
            pltpu.VMEM((RCH, D), jnp.float32),
            pltpu.VMEM((RCH, D), jnp.float32),
            pltpu.SemaphoreType.DMA,
            pltpu.SemaphoreType.DMA,
            pltpu.SemaphoreType.DMA,
            pltpu.SemaphoreType.DMA,
        ],
    )(_scatter_x_body)


def _scatter_x_body(routes_hbm, ranks_hbm, counts_hbm, x_hbm, xs_hbm,
                    off_v, rt_v, rk_v, dst2, rows0, rows1,
                    sin0, sin1, sout0, sout1):
    c = lax.axis_index("c")
    s = lax.axis_index("s")
    base_tok = (c * 16 + s) * TPT
    _dest_slots(counts_hbm, routes_hbm, ranks_hbm, off_v, rt_v, rk_v, dst2,
                base_tok)
    bufs = (rows0, rows1)
    sins = (sin0, sin1)
    souts = (sout0, sout1)
    loads = [None] * XCH
    stores = [None] * XCH
    loads[0] = pltpu.async_copy(x_hbm.at[pl.ds(base_tok, RCH)], rows0, sin0)
    for j in range(XCH):
        loads[j].wait()
        if j + 1 < XCH:
            if j >= 1:
                stores[j - 1].wait()
            loads[j + 1] = pltpu.async_copy(
                x_hbm.at[pl.ds(base_tok + RCH * (j + 1), RCH)],
                bufs[(j + 1) % 2], sins[(j + 1) % 2])
        stores[j] = pltpu.async_copy(bufs[j % 2], xs_hbm.at[dst2.at[j]],
                                     souts[j % 2])
    stores[XCH - 2].wait()
    stores[XCH - 1].wait()


def _mm_body(eid_ref, x_ref, w_ref, b_ref, y_ref):
    e = eid_ref[pl.program_id(0)]
    onehot_e = (lax.broadcasted_iota(jnp.int32, (E,), 0)
                == e).astype(jnp.float32)
    b_row = jnp.sum(b_ref[...] * onehot_e[:, None], axis=0)
    y_ref[...] = jnp.dot(x_ref[...], w_ref[e],
                         preferred_element_type=jnp.float32) + b_row[None, :]


def _grouped_mm(eid, xs, W, b):
    grid_spec = pltpu.PrefetchScalarGridSpec(
        num_scalar_prefetch=1,
        grid=(NB,),
        in_specs=[
            pl.BlockSpec((BM, D), lambda m, eid_ref: (m, 0)),
            pl.BlockSpec((E, D, O), lambda m, eid_ref: (0, 0, 0)),
            pl.BlockSpec((E, O), lambda m, eid_ref: (0, 0)),
        ],
        out_specs=pl.BlockSpec((BM, O), lambda m, eid_ref: (m, 0)),
    )
    return pl.pallas_call(
        _mm_body,
        grid_spec=grid_spec,
        out_shape=jax.ShapeDtypeStruct((PT, O), jnp.float32),
    )(eid, xs, W, b)


@functools.cache
def _build_gather_y():
    return functools.partial(
        pl.kernel,
        mesh=_sc_mesh(),
        compiler_params=_sc_params(),
        out_type=[
            jax.ShapeDtypeStruct((T, O), jnp.float32),
            jax.ShapeDtypeStruct((T, O), jnp.float32),
        ],
        scratch_types=[
            pltpu.VMEM((16,), jnp.int32),
            pltpu.VMEM((TPT,), jnp.int32),
            pltpu.VMEM((TPT,), jnp.int32),
            pltpu.VMEM((XCH, RCH), jnp.int32),
            pltpu.VMEM((RCH, O), jnp.float32),
            pltpu.VMEM((RCH, O), jnp.float32),
            pltpu.SemaphoreType.DMA,
            pltpu.SemaphoreType.DMA,
            pltpu.SemaphoreType.DMA,
            pltpu.SemaphoreType.DMA,
            pltpu.SemaphoreType.DMA,
            pltpu.SemaphoreType.DMA,
        ],
    )(_gather_y_body)


def _gather_y_body(routes_hbm, ranks_hbm, counts_hbm, ys_hbm,
                   out_hbm, out2_hbm,
                   off_v, rt_v, rk_v, dst2, rows0, rows1,
                   sin0, sin1, sout0, sout1, sout2_0, sout2_1):
    c = lax.axis_index("c")
    s = lax.axis_index("s")
    base_tok = (c * 16 + s) * TPT
    _dest_slots(counts_hbm, routes_hbm, ranks_hbm, off_v, rt_v, rk_v, dst2,
                base_tok)
    bufs = (rows0, rows1)
    sins = (sin0, sin1)
    souts = (sout0, sout1)
    souts2 = (sout2_0, sout2_1)
    loads = [None] * XCH
    stores = [None] * XCH
    stores2 = [None] * XCH
    loads[0] = pltpu.async_copy(ys_hbm.at[dst2.at[0]], rows0, sin0)
    for j in range(XCH):
        loads[j].wait()
        if j + 1 < XCH:
            if j >= 1:
                stores[j - 1].wait()
                stores2[j - 1].wait()
            loads[j + 1] = pltpu.async_copy(ys_hbm.at[dst2.at[j + 1]],
                                            bufs[(j + 1) % 2],
                                            sins[(j + 1) % 2])
        row_slice = pl.ds(base_tok + RCH * j, RCH)
        stores[j] = pltpu.async_copy(bufs[j % 2], out_hbm.at[row_slice],
                                     souts[j % 2])
        stores2[j] = pltpu.async_copy(bufs[j % 2], out2_hbm.at[row_slice],
                                      souts2[j % 2])
    stores[XCH - 2].wait()
    stores2[XCH - 2].wait()
    stores[XCH - 1].wait()
    stores2[XCH - 1].wait()


def kernel(x, Wr, br, W, b):
    probs, routes, ranks, counts16, eid, counts_i = _router(x, Wr, br)
    xs = _build_scatter_x()(routes, ranks, counts_i, x)
    ys = _grouped_mm(eid, xs, W, b)
    out, out2 = _build_gather_y()(routes, ranks, counts_i, ys)
    counts = counts16[:E]
    return (out, out2, probs, counts)
